# baseline (speedup 1.0000x reference)
import functools

import jax
import jax.numpy as jnp
from jax import lax
from jax.experimental import pallas as pl
from jax.experimental.pallas import tpu as pltpu
from jax.experimental.pallas import tpu_sc as plsc

DIM = 64
NW = 32
CHUNK = 128
NBUF = 8


@functools.lru_cache(maxsize=None)
def _build(rows):
    rows_per_w = rows // NW
    groups = rows_per_w // CHUNK
    mesh = plsc.VectorSubcoreMesh(core_axis_name="c", subcore_axis_name="s")

    @functools.partial(
        pl.kernel,
        mesh=mesh,
        out_type=jax.ShapeDtypeStruct((rows, DIM), jnp.float32),
        scratch_types=[
            pltpu.VMEM((groups, CHUNK), jnp.int32),
            pltpu.VMEM((NBUF, CHUNK, DIM), jnp.float32),
            pltpu.SemaphoreType.DMA,
            pltpu.SemaphoreType.DMA,
        ],
        compiler_params=pltpu.CompilerParams(use_tc_tiling_on_sc=False),
    )
    def gather_kernel(idx_hbm, table_hbm, out_hbm, idx_v, rows_v, gsem, wsem):
        wid = lax.axis_index("s") * 2 + lax.axis_index("c")
        base = wid * rows_per_w
        pltpu.sync_copy(idx_hbm.at[wid], idx_v)

        def body(i, carry):
            g0 = i * NBUF
            gets = [
                pltpu.async_copy(
                    table_hbm.at[idx_v.at[g0 + b]], rows_v.at[b], gsem)
                for b in range(NBUF)
            ]
            for c in gets:
                c.wait()
            puts = [
                pltpu.async_copy(
                    rows_v.at[b],
                    out_hbm.at[pl.ds(base + (g0 + b) * CHUNK, CHUNK)],
                    wsem)
                for b in range(NBUF)
            ]
            for c in puts:
                c.wait()
            return carry

        lax.fori_loop(0, groups // NBUF, body, 0)

    return gather_kernel


def kernel(indices, W):
    batch, hist = indices.shape
    rows = batch * hist
    idx = indices.reshape(NW, rows // (NW * CHUNK), CHUNK).astype(jnp.int32)
    out = _build(rows)(idx, W)
    return out.reshape(batch, hist, W.shape[1])


# speedup vs baseline: 1.0071x; 1.0071x over previous
import functools

import jax
import jax.numpy as jnp
from jax import lax
from jax.experimental import pallas as pl
from jax.experimental.pallas import tpu as pltpu
from jax.experimental.pallas import tpu_sc as plsc

DIM = 64
NW = 32
CHUNK = 128
GPG = 4


@functools.lru_cache(maxsize=None)
def _build(rows):
    rows_per_w = rows // NW
    groups = rows_per_w // CHUNK
    nsteps = groups // GPG
    mesh = plsc.VectorSubcoreMesh(core_axis_name="c", subcore_axis_name="s")

    @functools.partial(
        pl.kernel,
        mesh=mesh,
        out_type=jax.ShapeDtypeStruct((rows, DIM), jnp.float32),
        scratch_types=[
            pltpu.VMEM((groups, CHUNK), jnp.int32),
            pltpu.VMEM((2 * GPG, CHUNK, DIM), jnp.float32),
            pltpu.SemaphoreType.DMA,
            pltpu.SemaphoreType.DMA,
        ],
        compiler_params=pltpu.CompilerParams(use_tc_tiling_on_sc=False),
    )
    def gather_kernel(idx_hbm, table_hbm, out_hbm, idx_v, rows_v, gsem, wsem):
        wid = lax.axis_index("s") * 2 + lax.axis_index("c")
        base = wid * rows_per_w
        pltpu.sync_copy(idx_hbm.at[wid], idx_v)

        def fire_gathers(step, half):
            for j in range(GPG):
                pltpu.async_copy(
                    table_hbm.at[idx_v.at[step * GPG + j]],
                    rows_v.at[half + j], gsem)

        def fire_writes(step, half):
            for j in range(GPG):
                pltpu.async_copy(
                    rows_v.at[half + j],
                    out_hbm.at[pl.ds(base + (step * GPG + j) * CHUNK, CHUNK)],
                    wsem)

        def drain_gathers():
            for j in range(GPG):
                pltpu.make_async_copy(
                    table_hbm.at[pl.ds(0, CHUNK)], rows_v.at[j], gsem).wait()

        def drain_writes():
            for j in range(GPG):
                pltpu.make_async_copy(
                    rows_v.at[j], out_hbm.at[pl.ds(0, CHUNK)], wsem).wait()

        fire_gathers(0, 0)
        drain_gathers()
        fire_writes(0, 0)
        fire_gathers(1, GPG)

        def body(i, carry):
            half = (i % 2) * GPG
            other = ((i + 1) % 2) * GPG
            drain_gathers()
            drain_writes()
            fire_writes(i, half)
            fire_gathers(i + 1, other)
            return carry

        lax.fori_loop(1, nsteps - 1, body, 0)

        last_half = ((nsteps - 1) % 2) * GPG
        drain_gathers()
        drain_writes()
        fire_writes(nsteps - 1, last_half)
        drain_writes()

    return gather_kernel


def kernel(indices, W):
    batch, hist = indices.shape
    rows = batch * hist
    idx = indices.reshape(NW, rows // (NW * CHUNK), CHUNK).astype(jnp.int32)
    out = _build(rows)(idx, W)
    return out.reshape(batch, hist, W.shape[1])
